# baseline (speedup 1.0000x reference)
import dataclasses
import functools

import jax
import jax.numpy as jnp
from jax import lax
from jax.experimental import pallas as pl
from jax.experimental.pallas import tpu as pltpu
from jax.experimental.pallas import tpu_sc as plsc

DIMENSIONS = 10000
NUM_LEVELS = 21
N_GRAM_SIZE = 4
WINDOW = 256
NUM_CLASSES = 5
BATCH = 256
STRIDE = WINDOW // 5
N_SAMPLES = (WINDOW + STRIDE - 1) // STRIDE
N_ROWS = 4 * N_SAMPLES
LANES = 16
ROW_PAD = 24


def _sc_hist_body(x_hbm, part_hbm, idx_v, xrow_v, part_v):
    w = lax.axis_index("s") * 2 + lax.axis_index("c")

    @pl.when(w < N_ROWS)
    def _():
        c = w // N_SAMPLES
        s = w - c * N_SAMPLES
        base = s * (STRIDE * 4) + c
        lane = lax.iota(jnp.int32, LANES)
        for k in range(BATCH // LANES):
            idx_v[pl.ds(k * LANES, LANES)] = (lane + k * LANES) * (WINDOW * 4) + base
        pltpu.sync_copy(x_hbm.at[idx_v.at[pl.ds(0, 128)]], xrow_v.at[pl.ds(0, 128)])
        pltpu.sync_copy(x_hbm.at[idx_v.at[pl.ds(128, 128)]], xrow_v.at[pl.ds(128, 128)])

        zeros16 = jnp.zeros((LANES,), jnp.float32)
        for l in range(LANES):
            part_v[l, pl.ds(0, LANES)] = zeros16
            part_v[l, pl.ds(ROW_PAD - LANES, LANES)] = zeros16
        ones = jnp.ones((LANES,), jnp.float32)
        for k in range(BATCH // LANES):
            v = xrow_v[pl.ds(k * LANES, LANES)]
            plsc.addupdate_scatter(part_v, [lane, v], ones)
        pltpu.sync_copy(part_v, part_hbm.at[w])


def _roll1(a):
    return jnp.concatenate([a[:, DIMENSIONS - 1:], a[:, :DIMENSIONS - 1]], axis=1)


def _tc_body(part, iM1, iM2, iM3, iM4, CiM1, CiM2, CiM3, CiM4, W, out_ref):
    iM_refs = (iM1, iM2, iM3, iM4)
    CiM_refs = (CiM1, CiM2, CiM3, CiM4)

    counts24 = jnp.sum(part[...], axis=1)
    shv = None
    for c in range(4):
        T_c = iM_refs[c][:NUM_LEVELS, :] * CiM_refs[c][...]
        counts_c = counts24[c * N_SAMPLES:(c + 1) * N_SAMPLES, :NUM_LEVELS]
        part_mm = jax.lax.dot_general(
            counts_c, T_c, (((1,), (0,)), ((), ())),
            preferred_element_type=jnp.float32)
        shv = part_mm if shv is None else shv + part_mm

    rows = [shv[i:i + 1, :] for i in range(N_SAMPLES)]
    A = jnp.zeros_like(rows[0])
    for i in range(N_SAMPLES - N_GRAM_SIZE + 1):
        g = rows[i] + A
        for n in range(1, N_GRAM_SIZE):
            g = _roll1(g) * (rows[i + n] + A)
        A = A + g
    total = rows[0]
    for i in range(1, N_SAMPLES):
        total = total + rows[i]
    total = total + float(N_SAMPLES) * A

    enc = jnp.where(total > 0, 1.0, -1.0)
    out_ref[...] = jax.lax.dot_general(
        enc, W[...], (((1,), (1,)), ((), ())),
        preferred_element_type=jnp.float32)


@jax.jit
def _run(x, iM1, iM2, iM3, iM4, CiM1, CiM2, CiM3, CiM4, W):
    x_flat = x.reshape(BATCH * WINDOW * 4)

    cp = pltpu.CompilerParams()
    if "needs_layout_passes" in pltpu.CompilerParams.__dataclass_fields__:
        cp = dataclasses.replace(cp, needs_layout_passes=False)
    hist = pl.kernel(
        _sc_hist_body,
        compiler_params=cp,
        out_type=jax.ShapeDtypeStruct((N_ROWS, LANES, ROW_PAD), jnp.float32),
        mesh=plsc.VectorSubcoreMesh(core_axis_name="c", subcore_axis_name="s"),
        scratch_types=[
            pltpu.VMEM((BATCH,), jnp.int32),
            pltpu.VMEM((BATCH,), jnp.int32),
            pltpu.VMEM((LANES, ROW_PAD), jnp.float32),
        ],
    )
    part = hist(x_flat)

    full = lambda a: pl.BlockSpec(a.shape, lambda i: (0,) * a.ndim)
    iM_spec = pl.BlockSpec((ROW_PAD, DIMENSIONS), lambda i: (0, 0))
    out = pl.pallas_call(
        _tc_body,
        grid=(1,),
        out_shape=jax.ShapeDtypeStruct((1, NUM_CLASSES), jnp.float32),
        in_specs=[full(part), iM_spec, iM_spec, iM_spec, iM_spec,
                  full(CiM1), full(CiM2), full(CiM3), full(CiM4), full(W)],
        out_specs=pl.BlockSpec((1, NUM_CLASSES), lambda i: (0, 0)),
    )(part, iM1, iM2, iM3, iM4, CiM1, CiM2, CiM3, CiM4, W)
    return out.reshape(NUM_CLASSES)


def kernel(x, iM1, iM2, iM3, iM4, CiM1, CiM2, CiM3, CiM4, W):
    return _run(x, iM1, iM2, iM3, iM4, CiM1, CiM2, CiM3, CiM4, W)


# speedup vs baseline: 1.9914x; 1.9914x over previous
import functools

import jax
import jax.numpy as jnp
from jax.experimental import pallas as pl

DIMENSIONS = 10000
NUM_LEVELS = 21
N_GRAM_SIZE = 4
WINDOW = 256
NUM_CLASSES = 5
BATCH = 256
STRIDE = WINDOW // 5
N_SAMPLES = (WINDOW + STRIDE - 1) // STRIDE


def _roll1(a):
    return jnp.concatenate([a[:, DIMENSIONS - 1:], a[:, :DIMENSIONS - 1]], axis=1)


def _hd_kernel(x2d, iM1, iM2, iM3, iM4, CiM1, CiM2, CiM3, CiM4, W, out_ref):
    iM_refs = (iM1, iM2, iM3, iM4)
    CiM_refs = (CiM1, CiM2, CiM3, CiM4)

    xb = x2d[...].astype(jnp.bfloat16)
    j_iota = jax.lax.broadcasted_iota(jnp.int32, (4 * WINDOW, N_SAMPLES), 0)
    s_iota = jax.lax.broadcasted_iota(jnp.int32, (4 * WINDOW, N_SAMPLES), 1)

    shv = None
    for c in range(4):
        T_c = iM_refs[c][:NUM_LEVELS, :] * CiM_refs[c][...]
        S_c = (j_iota == s_iota * (STRIDE * 4) + c).astype(jnp.bfloat16)
        xsel = jax.lax.dot_general(
            xb, S_c, (((1,), (0,)), ((), ())),
            preferred_element_type=jnp.float32)
        cols = [
            jnp.sum((xsel == float(v)).astype(jnp.float32), axis=0, keepdims=True)
            for v in range(NUM_LEVELS)
        ]
        counts_c = jnp.concatenate(cols, axis=0)
        part = jax.lax.dot_general(
            counts_c, T_c, (((0,), (0,)), ((), ())),
            preferred_element_type=jnp.float32)
        shv = part if shv is None else shv + part

    rows = [shv[i:i + 1, :] for i in range(N_SAMPLES)]
    A = jnp.zeros_like(rows[0])
    for i in range(N_SAMPLES - N_GRAM_SIZE + 1):
        g = rows[i] + A
        for n in range(1, N_GRAM_SIZE):
            g = _roll1(g) * (rows[i + n] + A)
        A = A + g
    total = rows[0]
    for i in range(1, N_SAMPLES):
        total = total + rows[i]
    total = total + float(N_SAMPLES) * A

    enc = jnp.where(total > 0, 1.0, -1.0)
    out_ref[...] = jax.lax.dot_general(
        enc, W[...], (((1,), (1,)), ((), ())),
        preferred_element_type=jnp.float32)


@jax.jit
def _run(x, iM1, iM2, iM3, iM4, CiM1, CiM2, CiM3, CiM4, W):
    x2d = x.reshape(BATCH, 4 * WINDOW)
    full = lambda a: pl.BlockSpec(a.shape, lambda i: (0,) * a.ndim)
    iM_spec = pl.BlockSpec((24, DIMENSIONS), lambda i: (0, 0))
    out = pl.pallas_call(
        _hd_kernel,
        grid=(1,),
        out_shape=jax.ShapeDtypeStruct((1, NUM_CLASSES), jnp.float32),
        in_specs=[full(x2d), iM_spec, iM_spec, iM_spec, iM_spec,
                  full(CiM1), full(CiM2), full(CiM3), full(CiM4), full(W)],
        out_specs=pl.BlockSpec((1, NUM_CLASSES), lambda i: (0, 0)),
    )(x2d, iM1, iM2, iM3, iM4, CiM1, CiM2, CiM3, CiM4, W)
    return out.reshape(NUM_CLASSES)


def kernel(x, iM1, iM2, iM3, iM4, CiM1, CiM2, CiM3, CiM4, W):
    return _run(x, iM1, iM2, iM3, iM4, CiM1, CiM2, CiM3, CiM4, W)
